# baseline (speedup 1.0000x reference)
import functools
import jax
import jax.numpy as jnp
from jax import lax
from jax.experimental import pallas as pl
from jax.experimental.pallas import tpu as pltpu
from jax.experimental.pallas import tpu_sc as plsc

N = 32768
D = 512
B = 16
BN = 4096
NB = N // BN

NC = 2
NS = 16
NW = NC * NS
NT = N // NW
K = 64
NCHUNK = NT // K


def _reduce_kernel(x_ref, w_ref, b_ref, batch_ref, means_ref, acc_ref, cnt_ref):
    s = pl.program_id(0)

    @pl.when(s == 0)
    def _init():
        acc_ref[...] = jnp.zeros_like(acc_ref)
        cnt_ref[...] = jnp.zeros_like(cnt_ref)

    bvec = batch_ref[0, 0, :]
    seg_iota = lax.broadcasted_iota(jnp.int32, (BN, B), 1)
    onehot = (bvec[:, None] == seg_iota).astype(jnp.float32)

    xb = x_ref[...]
    logit = jnp.sum(xb * w_ref[0, :][None, :], axis=1, keepdims=True) + b_ref[0]
    weighted = xb * jax.nn.sigmoid(logit)
    acc_ref[...] += jnp.dot(onehot.T, weighted, preferred_element_type=jnp.float32)
    cnt_ref[0, :] += jnp.sum(onehot, axis=0)

    @pl.when(s == pl.num_programs(0) - 1)
    def _fin():
        inv = 1.0 / jnp.maximum(cnt_ref[0, :], 1.0)
        means_ref[...] = acc_ref[...] * inv[:, None]


def _sc_emit_kernel(x_hbm, means_hbm, batch_hbm, dsto_hbm, dste_hbm, out_hbm,
                    idx_v, dsto_v, dste_v, ctx_v, xrow_v, sem0, sem1):
    wid = lax.axis_index("s") * NC + lax.axis_index("c")
    tbase = wid * NT

    def chunk(i, _):
        base = tbase + i * K
        pltpu.sync_copy(batch_hbm.at[pl.ds(base, K)], idx_v)
        pltpu.sync_copy(dsto_hbm.at[pl.ds(base, K)], dsto_v)
        pltpu.sync_copy(dste_hbm.at[pl.ds(base, K)], dste_v)
        pltpu.async_copy(means_hbm.at[idx_v], ctx_v, sem0).wait()
        pltpu.sync_copy(x_hbm.at[pl.ds(base, K)], xrow_v)
        pltpu.async_copy(xrow_v, out_hbm.at[dste_v], sem0).wait()
        pltpu.async_copy(ctx_v, out_hbm.at[dsto_v], sem1).wait()
        return ()

    lax.fori_loop(0, NCHUNK, chunk, ())


def kernel(x, W, b, batch):
    batch32 = batch.astype(jnp.int32)
    batch_r = batch32.reshape(NB, 1, BN)
    w_row = W.reshape(1, D)

    means = pl.pallas_call(
        _reduce_kernel,
        grid=(NB,),
        in_specs=[
            pl.BlockSpec((BN, D), lambda s: (s, 0)),
            pl.BlockSpec((1, D), lambda s: (0, 0)),
            pl.BlockSpec(memory_space=pltpu.SMEM),
            pl.BlockSpec((1, 1, BN), lambda s: (s, 0, 0)),
        ],
        out_specs=pl.BlockSpec((B, D), lambda s: (0, 0)),
        out_shape=jax.ShapeDtypeStruct((B, D), jnp.float32),
        scratch_shapes=[
            pltpu.VMEM((B, D), jnp.float32),
            pltpu.VMEM((1, B), jnp.float32),
        ],
    )(x, w_row, b, batch_r)

    rows = jnp.arange(N, dtype=jnp.int32)
    dst_even = rows * 2
    dst_odd = rows * 2 + 1

    mesh = plsc.VectorSubcoreMesh(core_axis_name="c", subcore_axis_name="s")
    emit = functools.partial(
        pl.kernel,
        out_type=jax.ShapeDtypeStruct((2 * N, D), jnp.float32),
        mesh=mesh,
        scratch_types=[
            pltpu.VMEM((K,), jnp.int32),
            pltpu.VMEM((K,), jnp.int32),
            pltpu.VMEM((K,), jnp.int32),
            pltpu.VMEM((K, D), jnp.float32),
            pltpu.VMEM((K, D), jnp.float32),
            pltpu.SemaphoreType.DMA,
            pltpu.SemaphoreType.DMA,
        ],
    )(_sc_emit_kernel)

    out2 = emit(x, means, batch32, dst_odd, dst_even)
    return out2.reshape(N, 2 * D)


# speedup vs baseline: 7.7704x; 7.7704x over previous
import jax
import jax.numpy as jnp
from jax import lax
from jax.experimental import pallas as pl
from jax.experimental.pallas import tpu as pltpu

N = 32768
D = 512
B = 16
BN = 4096
NB = N // BN


def _attn_pool_kernel(x_ref, w_ref, b_ref, batch_ref, out_ref, acc_ref, cnt_ref):
    s = pl.program_id(0)
    nb = pl.num_programs(0) // 2

    @pl.when(s == 0)
    def _init():
        acc_ref[...] = jnp.zeros_like(acc_ref)
        cnt_ref[...] = jnp.zeros_like(cnt_ref)

    bvec = batch_ref[0, 0, :]
    seg_iota = lax.broadcasted_iota(jnp.int32, (BN, B), 1)
    onehot = (bvec[:, None] == seg_iota).astype(jnp.float32)

    @pl.when(s < nb)
    def _phase1():
        xb = x_ref[...]
        logit = jnp.sum(xb * w_ref[0, :][None, :], axis=1, keepdims=True) + b_ref[0]
        weighted = xb * jax.nn.sigmoid(logit)
        acc_ref[...] += jnp.dot(onehot.T, weighted,
                                preferred_element_type=jnp.float32)
        cnt_ref[0, :] += jnp.sum(onehot, axis=0)
        out_ref[...] = xb

    @pl.when(s >= nb)
    def _phase2():
        inv = 1.0 / jnp.maximum(cnt_ref[0, :], 1.0)
        means = acc_ref[...] * inv[:, None]
        out_ref[...] = jnp.dot(onehot, means,
                               preferred_element_type=jnp.float32)


def kernel(x, W, b, batch):
    batch32 = batch.astype(jnp.int32).reshape(NB, 1, BN)
    w_row = W.reshape(1, D)

    grid = (2 * NB,)
    out = pl.pallas_call(
        _attn_pool_kernel,
        grid=grid,
        in_specs=[
            pl.BlockSpec((BN, D), lambda s: (jnp.minimum(s, NB - 1), 0)),
            pl.BlockSpec((1, D), lambda s: (0, 0)),
            pl.BlockSpec(memory_space=pltpu.SMEM),
            pl.BlockSpec((1, 1, BN), lambda s: (lax.rem(s, NB), 0, 0)),
        ],
        out_specs=pl.BlockSpec((BN, D), lambda s: (lax.rem(s, NB), s // NB)),
        out_shape=jax.ShapeDtypeStruct((N, 2 * D), jnp.float32),
        scratch_shapes=[
            pltpu.VMEM((B, D), jnp.float32),
            pltpu.VMEM((1, B), jnp.float32),
        ],
    )(x, w_row, b, batch32)
    return out
